# baseline (speedup 1.0000x reference)
import functools

import jax
import jax.numpy as jnp
from jax import lax
from jax.experimental import pallas as pl
from jax.experimental.pallas import tpu as pltpu
from jax.experimental.pallas import tpu_sc as plsc

B, D = 16384, 128
NC, NS, L = 2, 16, 16
NW = NC * NS
B_SC = 2048
RPW = B_SC // NW
GRP = RPW // L
B_TC = B - B_SC
BLK = 3584
MARGIN = 1.0
EPS = 1e-6


def _sqrt16(x):
    i = plsc.bitcast(x, jnp.int32)
    y = plsc.bitcast(jnp.int32(0x5F3759DF) - (i >> 1), jnp.float32)
    for _ in range(3):
        y = y * (1.5 - 0.5 * x * y * y)
    return jnp.where(x > 0.0, x * y, 0.0)


@functools.partial(
    pl.kernel,
    out_type=jax.ShapeDtypeStruct((NW, L), jnp.float32),
    mesh=plsc.VectorSubcoreMesh(core_axis_name="c", subcore_axis_name="s"),
    compiler_params=pltpu.CompilerParams(needs_layout_passes=False),
    scratch_types=[
        pltpu.VMEM((RPW * D,), jnp.float32),
        pltpu.VMEM((RPW * D,), jnp.float32),
        pltpu.VMEM((RPW * D,), jnp.float32),
        pltpu.VMEM((L,), jnp.float32),
    ],
)
def _sc_partials(a_hbm, p_hbm, n_hbm, out_hbm, a_v, p_v, n_v, l_v):
    wid = lax.axis_index("s") * NC + lax.axis_index("c")
    e0 = (B_TC + wid * RPW) * D
    pltpu.sync_copy(a_hbm.at[pl.ds(e0, RPW * D)], a_v)
    pltpu.sync_copy(p_hbm.at[pl.ds(e0, RPW * D)], p_v)
    pltpu.sync_copy(n_hbm.at[pl.ds(e0, RPW * D)], n_v)

    def group_body(g, acc):
        lane = lax.iota(jnp.int32, L)
        rows = (g * L + lane) * D
        rl = rows + lane
        ap = [jnp.zeros((L,), jnp.float32) for _ in range(4)]
        an = [jnp.zeros((L,), jnp.float32) for _ in range(4)]
        for c16 in range(D):
            if c16 <= D - L:
                idx = rl + c16
            else:
                wrap = jnp.where(lane >= D - c16, D, 0)
                idx = rl + c16 - wrap
            av = plsc.load_gather(a_v, [idx])
            pv = plsc.load_gather(p_v, [idx])
            nv = plsc.load_gather(n_v, [idx])
            k = c16 & 3
            ave = av + EPS
            t1 = ave - pv
            ap[k] = ap[k] + t1 * t1
            t2 = ave - nv
            an[k] = an[k] + t2 * t2
        d_ap = _sqrt16((ap[0] + ap[1]) + (ap[2] + ap[3]))
        d_an = _sqrt16((an[0] + an[1]) + (an[2] + an[3]))
        return acc + jnp.maximum(d_ap - d_an + MARGIN, 0.0)

    lacc = lax.fori_loop(0, GRP, group_body, jnp.zeros((L,), jnp.float32))
    l_v[...] = lacc
    pltpu.sync_copy(l_v, out_hbm.at[wid])


def _tc_body(a_ref, p_ref, n_ref, out_ref):
    pid = pl.program_id(0)
    a = a_ref[...]
    t1 = a - p_ref[...] + EPS
    t2 = a - n_ref[...] + EPS
    ones = jnp.ones((D, 8), jnp.float32)
    ap = jax.lax.dot_general(t1 * t1, ones, (((1,), (0,)), ((), ())),
                             precision=jax.lax.Precision.HIGHEST)
    an = jax.lax.dot_general(t2 * t2, ones, (((1,), (0,)), ((), ())),
                             precision=jax.lax.Precision.HIGHEST)
    h = jnp.maximum(jnp.sqrt(ap) - jnp.sqrt(an) + MARGIN, 0.0)
    s = (jnp.sum(h) * 0.125).reshape(1, 1)

    @pl.when(pid == 0)
    def _():
        out_ref[...] = s

    @pl.when(pid != 0)
    def _():
        out_ref[...] += s


_tc_partial = pl.pallas_call(
    _tc_body,
    grid=(B_TC // BLK,),
    in_specs=[
        pl.BlockSpec((BLK, D), lambda i: (i, 0)),
        pl.BlockSpec((BLK, D), lambda i: (i, 0)),
        pl.BlockSpec((BLK, D), lambda i: (i, 0)),
    ],
    out_specs=pl.BlockSpec((1, 1), lambda i: (0, 0)),
    out_shape=jax.ShapeDtypeStruct((1, 1), jnp.float32),
)


def kernel(anchor, positive, negative, anchor_label, positive_label,
           negative_label, eval_mode):
    tc_part = _tc_partial(anchor, positive, negative)
    sc_part = _sc_partials(
        anchor.reshape(-1), positive.reshape(-1), negative.reshape(-1)
    )
    loss = (jnp.sum(sc_part) + tc_part[0, 0]) * (1.0 / B)
    return jnp.nan_to_num(loss, nan=0.0)


# speedup vs baseline: 1.3663x; 1.3663x over previous
import functools

import jax
import jax.numpy as jnp
from jax import lax
from jax.experimental import pallas as pl
from jax.experimental.pallas import tpu as pltpu
from jax.experimental.pallas import tpu_sc as plsc

B, D = 16384, 128
NC, NS, L = 2, 16, 16
NW = NC * NS
B_SC = 2048
RPW = B_SC // NW
GRP = RPW // L
B_TC = B - B_SC
BLK = 3584
MARGIN = 1.0
EPS = 1e-6


def _sqrt16(x):
    i = plsc.bitcast(x, jnp.int32)
    y = plsc.bitcast(jnp.int32(0x5F3759DF) - (i >> 1), jnp.float32)
    for _ in range(3):
        y = y * (1.5 - 0.5 * x * y * y)
    return jnp.where(x > 0.0, x * y, 0.0)


@functools.partial(
    pl.kernel,
    out_type=jax.ShapeDtypeStruct((NW, L), jnp.float32),
    mesh=plsc.VectorSubcoreMesh(core_axis_name="c", subcore_axis_name="s"),
    compiler_params=pltpu.CompilerParams(needs_layout_passes=False),
    scratch_types=[
        pltpu.VMEM((RPW * D,), jnp.float32),
        pltpu.VMEM((RPW * D,), jnp.float32),
        pltpu.VMEM((RPW * D,), jnp.float32),
        pltpu.VMEM((L,), jnp.float32),
    ],
)
def _sc_partials(a_hbm, p_hbm, n_hbm, out_hbm, a_v, p_v, n_v, l_v):
    wid = lax.axis_index("s") * NC + lax.axis_index("c")
    e0 = (B_TC + wid * RPW) * D
    pltpu.sync_copy(a_hbm.at[pl.ds(e0, RPW * D)], a_v)
    pltpu.sync_copy(p_hbm.at[pl.ds(e0, RPW * D)], p_v)
    pltpu.sync_copy(n_hbm.at[pl.ds(e0, RPW * D)], n_v)

    def group_body(g, acc):
        lane = lax.iota(jnp.int32, L)
        rows = (g * L + lane) * D
        rl = rows + lane
        ap = [jnp.zeros((L,), jnp.float32) for _ in range(4)]
        an = [jnp.zeros((L,), jnp.float32) for _ in range(4)]
        for c16 in range(D):
            if c16 <= D - L:
                idx = rl + c16
            else:
                wrap = jnp.where(lane >= D - c16, D, 0)
                idx = rl + c16 - wrap
            av = plsc.load_gather(a_v, [idx])
            pv = plsc.load_gather(p_v, [idx])
            nv = plsc.load_gather(n_v, [idx])
            k = c16 & 3
            ave = av + EPS
            t1 = ave - pv
            ap[k] = ap[k] + t1 * t1
            t2 = ave - nv
            an[k] = an[k] + t2 * t2
        d_ap = _sqrt16((ap[0] + ap[1]) + (ap[2] + ap[3]))
        d_an = _sqrt16((an[0] + an[1]) + (an[2] + an[3]))
        return acc + jnp.maximum(d_ap - d_an + MARGIN, 0.0)

    lacc = lax.fori_loop(0, GRP, group_body, jnp.zeros((L,), jnp.float32))
    l_v[...] = lacc
    pltpu.sync_copy(l_v, out_hbm.at[wid])


def _tc_body(a_ref, p_ref, n_ref, out_ref):
    pid = pl.program_id(0)
    a = a_ref[...]
    t1 = a - p_ref[...] + EPS
    t2 = a - n_ref[...] + EPS
    ones = jnp.ones((D, 8), jnp.float32)
    ap = jax.lax.dot_general(t1 * t1, ones, (((1,), (0,)), ((), ())))
    an = jax.lax.dot_general(t2 * t2, ones, (((1,), (0,)), ((), ())))
    h = jnp.maximum(jnp.sqrt(ap) - jnp.sqrt(an) + MARGIN, 0.0)
    s = (jnp.sum(h) * 0.125).reshape(1, 1)

    @pl.when(pid == 0)
    def _():
        out_ref[...] = s

    @pl.when(pid != 0)
    def _():
        out_ref[...] += s


_tc_partial = pl.pallas_call(
    _tc_body,
    grid=(B_TC // BLK,),
    in_specs=[
        pl.BlockSpec((BLK, D), lambda i: (i, 0)),
        pl.BlockSpec((BLK, D), lambda i: (i, 0)),
        pl.BlockSpec((BLK, D), lambda i: (i, 0)),
    ],
    out_specs=pl.BlockSpec((1, 1), lambda i: (0, 0)),
    out_shape=jax.ShapeDtypeStruct((1, 1), jnp.float32),
)


def kernel(anchor, positive, negative, anchor_label, positive_label,
           negative_label, eval_mode):
    tc_part = _tc_partial(anchor, positive, negative)
    sc_part = _sc_partials(
        anchor.reshape(-1), positive.reshape(-1), negative.reshape(-1)
    )
    loss = (jnp.sum(sc_part) + tc_part[0, 0]) * (1.0 / B)
    return jnp.nan_to_num(loss, nan=0.0)
